# baseline (speedup 1.0000x reference)
import functools

import jax
import jax.numpy as jnp
from jax.experimental import pallas as pl
from jax.experimental.pallas import tpu as pltpu
from jax.experimental.pallas import tpu_sc as plsc


def _vq_body(x_ref, w_ref, idx_ref, e_ref):
    nb = x_ref.shape[0]
    w = w_ref[...]
    K = w.shape[0]
    P = x_ref.shape[2]
    w2 = jnp.sum(w * w, axis=1)
    for i in range(nb):
        x = x_ref[i]
        xw = jax.lax.dot_general(w, x, (((1,), (0,)), ((), ())),
                                 preferred_element_type=jnp.float32)
        x2 = jnp.sum(x * x, axis=0)
        d = (x2[None, :] + w2[:, None]) - 2.0 * xw
        m = jnp.min(d, axis=0)
        kk = jax.lax.broadcasted_iota(jnp.int32, d.shape, 0)
        idx = jnp.min(jnp.where(d == m[None, :], kk, K), axis=0)
        idx_ref[i] = idx
        idx_col = jnp.transpose(idx.reshape(1, P))
        p_iota = jax.lax.broadcasted_iota(jnp.int32, (P, K), 1)
        e_ref[pl.ds(i * P, P), :] = (p_iota == idx_col).astype(jnp.float32)


def _sc_gather_body(wt_hbm, idx_hbm, q_hbm, wt_v, idx_v, out_v):
    b = jax.lax.axis_index("s") * 2 + jax.lax.axis_index("c")
    pltpu.sync_copy(wt_hbm, wt_v)
    pltpu.sync_copy(idx_hbm.at[b], idx_v)

    def chunk(j, carry):
        idx16 = idx_v[pl.ds(j * 16, 16)]
        for ch in range(64):
            ch16 = jnp.full((16,), ch, jnp.int32)
            out_v[ch, pl.ds(j * 16, 16)] = plsc.load_gather(
                wt_v, [ch16, idx16])
        return carry

    jax.lax.fori_loop(0, 64, chunk, 0)
    pltpu.sync_copy(out_v, q_hbm.at[b])


@functools.partial(jax.jit, static_argnames=("interpret",))
def kernel(inputs, W, interpret=False):
    B, C, H, Wd = inputs.shape
    P = H * Wd
    K = W.shape[0]
    x3 = inputs.reshape(B, C, P)
    NB = 8
    idx2d, e = pl.pallas_call(
        _vq_body,
        grid=(B // NB,),
        in_specs=[
            pl.BlockSpec((NB, C, P), lambda b: (b, 0, 0)),
            pl.BlockSpec((K, C), lambda b: (0, 0)),
        ],
        out_specs=[
            pl.BlockSpec((NB, P), lambda b: (b, 0)),
            pl.BlockSpec((NB * P, K), lambda b: (b, 0)),
        ],
        out_shape=[
            jax.ShapeDtypeStruct((B, P), jnp.int32),
            jax.ShapeDtypeStruct((B * P, K), jnp.float32),
        ],
        interpret=interpret,
    )(x3, W)

    wt = W.T
    sc_gather = pl.kernel(
        _sc_gather_body,
        out_type=jax.ShapeDtypeStruct((B, C, P), jnp.float32),
        mesh=plsc.VectorSubcoreMesh(core_axis_name="c", subcore_axis_name="s"),
        compiler_params=pltpu.CompilerParams(needs_layout_passes=False),
        scratch_types=[
            pltpu.VMEM((C, K), jnp.float32),
            pltpu.VMEM((P,), jnp.int32),
            pltpu.VMEM((C, P), jnp.float32),
        ],
    )
    q3 = sc_gather(wt, idx2d)
    return q3.reshape(B, C, H, Wd), e
